# Initial kernel scaffold; baseline (speedup 1.0000x reference)
import jax
import jax.numpy as jnp
from jax.experimental import pallas as pl


def kernel(x):
    raise NotImplementedError("write your pallas kernel here")


# speedup vs baseline: 64.6623x; 64.6623x over previous
import functools

import jax
import jax.numpy as jnp
import numpy as np
from jax.experimental import pallas as pl
from jax.experimental.pallas import tpu as pltpu

_THRESHOLD = 0.05
_CACHELINE = 64
_IMAX = np.int32(0x7FFFFFFF)
_VBITS = 25


def _cluster_body(x_ref, o_ref, bv_ref, bvk_ref):
    inf = jnp.float32(jnp.inf)
    bv_ref[:] = jnp.full(bv_ref.shape, inf, jnp.float32)
    bvk_ref[:] = jnp.full(bvk_ref.shape, _IMAX, jnp.int32)
    for i in range(_CACHELINE):
        v = x_ref[i : i + 1, :]
        m = jnp.abs(bv_ref[:] - v) < _THRESHOLD
        key = jnp.where(m, bvk_ref[:], _IMAX)
        kmin = jnp.min(key, axis=0, keepdims=True)
        found = kmin < _IMAX
        mv_bits = (kmin & ((1 << _VBITS) - 1)) << (32 - _VBITS)
        mv = jax.lax.bitcast_convert_type(mv_bits, jnp.float32)
        o_ref[i : i + 1, :] = jnp.where(found, mv, v)
        vbits = jax.lax.bitcast_convert_type(v, jnp.int32)
        packed = jnp.int32(i << _VBITS) | jax.lax.shift_right_logical(
            vbits, 32 - _VBITS
        )
        bv_ref[i : i + 1, :] = jnp.where(found, inf, v)
        bvk_ref[i : i + 1, :] = jnp.where(found, _IMAX, packed)


@functools.partial(jax.jit, static_argnums=(1,))
def _cluster_flat(xt, lanes):
    n = xt.shape[1]
    grid = n // lanes
    return pl.pallas_call(
        _cluster_body,
        grid=(grid,),
        in_specs=[pl.BlockSpec((_CACHELINE, lanes), lambda i: (0, i))],
        out_specs=pl.BlockSpec((_CACHELINE, lanes), lambda i: (0, i)),
        out_shape=jax.ShapeDtypeStruct((_CACHELINE, n), jnp.float32),
        scratch_shapes=[
            pltpu.VMEM((_CACHELINE, lanes), jnp.float32),
            pltpu.VMEM((_CACHELINE, lanes), jnp.int32),
        ],
    )(xt)


def kernel(x):
    shape = x.shape
    flat = x.reshape(-1)
    total = flat.shape[0]
    n_full = (total // _CACHELINE) * _CACHELINE
    n_chunks = n_full // _CACHELINE
    lanes = next(c for c in (512, 256, 128, 64, 32, 16, 8, 4, 2, 1)
                 if n_chunks % c == 0)
    xt = flat[:n_full].reshape(n_chunks, _CACHELINE).T
    out_t = _cluster_flat(xt, lanes)
    out = out_t.T.reshape(-1)
    if n_full != total:
        out = jnp.concatenate([out, flat[n_full:]])
    return out.reshape(shape)


# speedup vs baseline: 89.9149x; 1.3905x over previous
import functools

import jax
import jax.numpy as jnp
import numpy as np
from jax.experimental import pallas as pl
from jax.experimental.pallas import tpu as pltpu

_THRESHOLD = 0.05
_CACHELINE = 64
_VBITS = 24


def _cluster_body(x_ref, o_ref, xt_ref, bv_ref, pk_ref, ot_ref):
    inf = jnp.float32(jnp.inf)
    xt = x_ref[0].T
    xt_ref[:] = xt
    rows = jax.lax.broadcasted_iota(jnp.int32, xt.shape, 0)
    xbits = jax.lax.bitcast_convert_type(xt, jnp.int32)
    packed = ((1 << 30) | (rows << _VBITS)) | jax.lax.shift_right_logical(
        xbits, 32 - _VBITS
    )
    pk_ref[:] = jax.lax.bitcast_convert_type(packed, jnp.float32)
    bv_ref[:] = jnp.full(bv_ref.shape, inf, jnp.float32)
    for i in range(_CACHELINE):
        v = xt_ref[i : i + 1, :]
        if i == 0:
            ot_ref[i : i + 1, :] = v
            bv_ref[i : i + 1, :] = v
            continue
        nrows = min(-(-i // 8) * 8, _CACHELINE)
        m = jnp.abs(bv_ref[0:nrows, :] - v) < _THRESHOLD
        key = jnp.where(m, pk_ref[0:nrows, :], inf)
        kmin = jnp.min(key, axis=0, keepdims=True)
        found = kmin < inf
        ki = jax.lax.bitcast_convert_type(kmin, jnp.int32)
        mv_bits = (ki & ((1 << _VBITS) - 1)) << (32 - _VBITS)
        mv = jax.lax.bitcast_convert_type(mv_bits, jnp.float32)
        ot_ref[i : i + 1, :] = jnp.where(found, mv, v)
        bv_ref[i : i + 1, :] = jnp.where(found, inf, v)
    o_ref[0] = ot_ref[:].T


@functools.partial(jax.jit, static_argnums=(1,))
def _cluster_flat(xc, lanes):
    n = xc.shape[0]
    grid = n // lanes
    x3 = xc.reshape(grid, lanes, _CACHELINE)
    out = pl.pallas_call(
        _cluster_body,
        grid=(grid,),
        in_specs=[pl.BlockSpec((1, lanes, _CACHELINE), lambda i: (i, 0, 0))],
        out_specs=pl.BlockSpec((1, lanes, _CACHELINE), lambda i: (i, 0, 0)),
        out_shape=jax.ShapeDtypeStruct((grid, lanes, _CACHELINE), jnp.float32),
        scratch_shapes=[
            pltpu.VMEM((_CACHELINE, lanes), jnp.float32),
            pltpu.VMEM((_CACHELINE, lanes), jnp.float32),
            pltpu.VMEM((_CACHELINE, lanes), jnp.float32),
            pltpu.VMEM((_CACHELINE, lanes), jnp.float32),
        ],
    )(x3)
    return out.reshape(n, _CACHELINE)


def kernel(x):
    shape = x.shape
    flat = x.reshape(-1)
    total = flat.shape[0]
    n_full = (total // _CACHELINE) * _CACHELINE
    n_chunks = n_full // _CACHELINE
    lanes = next(c for c in (512, 256, 128, 64, 32, 16, 8, 4, 2, 1)
                 if n_chunks % c == 0)
    out = _cluster_flat(flat[:n_full].reshape(n_chunks, _CACHELINE), lanes)
    out = out.reshape(-1)
    if n_full != total:
        out = jnp.concatenate([out, flat[n_full:]])
    return out.reshape(shape)


# speedup vs baseline: 96.4150x; 1.0723x over previous
import functools

import jax
import jax.numpy as jnp
import numpy as np
from jax.experimental import pallas as pl
from jax.experimental.pallas import tpu as pltpu

_THRESHOLD = 0.05
_CACHELINE = 64
_VBITS = 24


def _cluster_body(x_ref, o_ref, xt_ref, bv_ref, pk_ref, ot_ref):
    inf = jnp.float32(jnp.inf)
    xt = x_ref[0].T
    xt_ref[:] = xt
    rows = jax.lax.broadcasted_iota(jnp.int32, xt.shape, 0)
    xbits = jax.lax.bitcast_convert_type(xt, jnp.int32)
    packed = ((1 << 30) | (rows << _VBITS)) | jax.lax.shift_right_logical(
        xbits, 32 - _VBITS
    )
    pk_ref[:] = jax.lax.bitcast_convert_type(packed, jnp.float32)
    bv_ref[:] = jnp.full(bv_ref.shape, inf, jnp.float32)
    lanes = bv_ref.shape[1]
    half = lanes // 2

    def step(i, lo, hi):
        v = xt_ref[i : i + 1, lo:hi]
        if i == 0:
            ot_ref[i : i + 1, lo:hi] = v
            bv_ref[i : i + 1, lo:hi] = v
            return
        nrows = min(-(-i // 8) * 8, _CACHELINE)
        m = jnp.abs(bv_ref[0:nrows, lo:hi] - v) < _THRESHOLD
        key = jnp.where(m, pk_ref[0:nrows, lo:hi], inf)
        kmin = jnp.min(key, axis=0, keepdims=True)
        found = kmin < inf
        ki = jax.lax.bitcast_convert_type(kmin, jnp.int32)
        mv_bits = (ki & ((1 << _VBITS) - 1)) << (32 - _VBITS)
        mv = jax.lax.bitcast_convert_type(mv_bits, jnp.float32)
        ot_ref[i : i + 1, lo:hi] = jnp.where(found, mv, v)
        bv_ref[i : i + 1, lo:hi] = jnp.where(found, inf, v)

    for i in range(_CACHELINE):
        step(i, 0, half)
        step(i, half, lanes)
    o_ref[0] = ot_ref[:].T


@functools.partial(jax.jit, static_argnums=(1,))
def _cluster_flat(xc, lanes):
    n = xc.shape[0]
    grid = n // lanes
    x3 = xc.reshape(grid, lanes, _CACHELINE)
    out = pl.pallas_call(
        _cluster_body,
        grid=(grid,),
        in_specs=[pl.BlockSpec((1, lanes, _CACHELINE), lambda i: (i, 0, 0))],
        out_specs=pl.BlockSpec((1, lanes, _CACHELINE), lambda i: (i, 0, 0)),
        out_shape=jax.ShapeDtypeStruct((grid, lanes, _CACHELINE), jnp.float32),
        scratch_shapes=[
            pltpu.VMEM((_CACHELINE, lanes), jnp.float32),
            pltpu.VMEM((_CACHELINE, lanes), jnp.float32),
            pltpu.VMEM((_CACHELINE, lanes), jnp.float32),
            pltpu.VMEM((_CACHELINE, lanes), jnp.float32),
        ],
    )(x3)
    return out.reshape(n, _CACHELINE)


def kernel(x):
    shape = x.shape
    flat = x.reshape(-1)
    total = flat.shape[0]
    n_full = (total // _CACHELINE) * _CACHELINE
    n_chunks = n_full // _CACHELINE
    lanes = next(c for c in (1024, 512, 256, 128, 64, 32, 16, 8, 4, 2, 1)
                 if n_chunks % c == 0)
    out = _cluster_flat(flat[:n_full].reshape(n_chunks, _CACHELINE), lanes)
    out = out.reshape(-1)
    if n_full != total:
        out = jnp.concatenate([out, flat[n_full:]])
    return out.reshape(shape)
